# baseline (speedup 1.0000x reference)
import functools

import jax
import jax.numpy as jnp
from jax import lax
from jax.experimental import pallas as pl
from jax.experimental.pallas import tpu as pltpu
from jax.experimental.pallas import tpu_sc as plsc

B = 100
N = 1000
E = 128

_GATHER_PAD = 128
_N_WORKERS_USED = 16
_ROWS_PER_WORKER = 8

_N_CHUNKS = 25
_NB = N // _N_CHUNKS


def _sc_gather(feat2d, flat_idx):
    mesh = plsc.VectorSubcoreMesh(core_axis_name="c", subcore_axis_name="s")

    @functools.partial(
        pl.kernel,
        out_type=jax.ShapeDtypeStruct((_GATHER_PAD, E), jnp.float32),
        mesh=mesh,
        scratch_types=[
            pltpu.VMEM((_ROWS_PER_WORKER,), jnp.int32),
            pltpu.VMEM((_ROWS_PER_WORKER, E), jnp.float32),
            pltpu.SemaphoreType.DMA,
        ],
    )
    def gather_kernel(feat_hbm, idx_hbm, out_hbm, idx_v, rows_v, sem):
        wid = lax.axis_index("s") * 2 + lax.axis_index("c")

        @pl.when(wid < _N_WORKERS_USED)
        def _():
            base = wid * _ROWS_PER_WORKER
            pltpu.sync_copy(idx_hbm.at[pl.ds(base, _ROWS_PER_WORKER)], idx_v)
            pltpu.async_copy(feat_hbm.at[idx_v], rows_v, sem).wait()
            pltpu.sync_copy(rows_v, out_hbm.at[pl.ds(base, _ROWS_PER_WORKER)])

    return gather_kernel(feat2d, flat_idx)


def _tc_body(feat_ref, cur_ref, act_ref, stc_ref, w5a_ref, w5b_ref, b5_ref,
             w6_ref, b6_ref, w7_ref, b7_ref, w8_ref, b8_ref, w9_ref, b9_ref,
             q_ref, acc_ref):
    i = pl.program_id(0)

    @pl.when(i == 0)
    def _():
        acc_ref[...] = jnp.zeros_like(acc_ref)

    acc_ref[...] += jnp.sum(feat_ref[0], axis=1)

    @pl.when(i == _N_CHUNKS - 1)
    def _():
        feat_sum = acc_ref[...]
        h1 = jnp.dot(feat_sum, w6_ref[...],
                     preferred_element_type=jnp.float32) + b6_ref[...]
        h2 = (jnp.dot(cur_ref[...], w7_ref[...],
                      preferred_element_type=jnp.float32) + b7_ref[...]
              + act_ref[...] * w8_ref[...] + b8_ref[...]
              + stc_ref[...] * w9_ref[...] + b9_ref[...])
        q = (jnp.dot(jnp.maximum(h1, 0.0), w5a_ref[...],
                     preferred_element_type=jnp.float32)
             + jnp.dot(jnp.maximum(h2, 0.0), w5b_ref[...],
                       preferred_element_type=jnp.float32)
             + b5_ref[...])
        q_ref[...] = q


def _tc_compute(feat, cur_feat, action, state_c, w5a, w5b, b5, w6, b6,
                w7, b7, w8, b8, w9, b9):
    full = lambda shape: pl.BlockSpec(shape, lambda i: (0,) * len(shape))
    return pl.pallas_call(
        _tc_body,
        grid=(_N_CHUNKS,),
        in_specs=[
            pl.BlockSpec((1, B, _NB, E), lambda i: (0, 0, i, 0)),
            full((B, E)),
            full((B, 1)),
            full((B, 1)),
            full((E, 1)),
            full((E, 1)),
            full((1, 1)),
            full((E, E)),
            full((1, E)),
            full((E, E)),
            full((1, E)),
            full((1, E)),
            full((1, E)),
            full((1, E)),
            full((1, E)),
        ],
        out_specs=pl.BlockSpec((B, 1), lambda i: (0, 0)),
        out_shape=jax.ShapeDtypeStruct((B, 1), jnp.float32),
        scratch_shapes=[pltpu.VMEM((B, E), jnp.float32)],
    )(feat.reshape(1, B, N, E), cur_feat, action, state_c, w5a, w5b, b5,
      w6, b6, w7, b7, w8, b8, w9, b9)


def kernel(feat, cur_node, action, state_c, W5, b5, W6, b6, W7, b7, W8, b8,
           W9, b9):
    feat2d = feat.reshape(B * N, E)
    flat_idx = jnp.arange(B, dtype=jnp.int32) * N + cur_node.astype(jnp.int32)
    flat_idx = jnp.concatenate(
        [flat_idx, jnp.zeros((_GATHER_PAD - B,), jnp.int32)])
    cur_feat = _sc_gather(feat2d, flat_idx)[:B]
    return _tc_compute(
        feat, cur_feat, action, state_c,
        W5[:E], W5[E:], b5.reshape(1, 1),
        W6, b6.reshape(1, E), W7, b7.reshape(1, E),
        W8.reshape(1, E), b8.reshape(1, E), W9.reshape(1, E),
        b9.reshape(1, E))
